# baseline (speedup 1.0000x reference)
import functools

import jax
import jax.numpy as jnp
from jax import lax
from jax.experimental import pallas as pl
from jax.experimental.pallas import tpu as pltpu
from jax.experimental.pallas import tpu_sc as plsc


def kernel(input_words, output_words, in_embed, out_embed):
    B, = input_words.shape
    V, D = in_embed.shape

    info = plsc.get_sparse_core_info()
    nc, ns, L = info.num_cores, info.num_subcores, info.num_lanes
    nw = nc * ns
    bpw = B // nw

    oe3 = out_embed.reshape(V // 8, 8, D)

    mesh = plsc.VectorSubcoreMesh(core_axis_name="c", subcore_axis_name="s")

    @functools.partial(
        pl.kernel,
        mesh=mesh,
        out_type=jax.ShapeDtypeStruct((2, B // 8, 8, D), jnp.float32),
        scratch_types=[
            pltpu.VMEM((bpw,), jnp.int32),
            pltpu.VMEM((bpw,), jnp.int32),
            pltpu.VMEM((bpw // 8, 8, D), jnp.float32),
            pltpu.SemaphoreType.DMA,
        ],
    )
    def _gather2(iw_hbm, ow_hbm, ie_hbm, oe_hbm, out_hbm,
                 idx0, idx1, buf, sem):
        wid = lax.axis_index("s") * nc + lax.axis_index("c")
        base = wid * bpw
        pltpu.sync_copy(iw_hbm.at[pl.ds(base, bpw)], idx0)
        pltpu.sync_copy(ow_hbm.at[pl.ds(base, bpw)], idx1)

        def issue2d(k, _):
            v = idx0[pl.ds(k * L, L)]
            for j in range(L):
                r = k * L + j
                pltpu.async_copy(
                    ie_hbm.at[pl.ds(v[j], 1)],
                    buf.at[r // 8, pl.ds(r % 8, 1)],
                    sem)
            return 0

        def issue3d(k, _):
            v = idx1[pl.ds(k * L, L)]
            tv = v >> 3
            sv = v & 7
            for j in range(L):
                r = k * L + j
                pltpu.async_copy(
                    oe_hbm.at[tv[j], pl.ds(sv[j], 1)],
                    buf.at[r // 8, pl.ds(r % 8, 1)],
                    sem)
            return 0

        for issue, out_row in ((issue2d, 0), (issue3d, 1)):
            lax.fori_loop(0, bpw // L, issue, 0)
            pltpu.make_async_copy(
                oe_hbm.at[pl.ds(0, bpw // 8)], buf, sem).wait()
            pltpu.sync_copy(
                buf, out_hbm.at[out_row, pl.ds(base // 8, bpw // 8)])

    out = _gather2(input_words, output_words, in_embed, oe3)
    return out.reshape(2, B, D)


# speedup vs baseline: 1.0827x; 1.0827x over previous
import functools

import jax
import jax.numpy as jnp
from jax import lax
from jax.experimental import pallas as pl
from jax.experimental.pallas import tpu as pltpu
from jax.experimental.pallas import tpu_sc as plsc


def kernel(input_words, output_words, in_embed, out_embed):
    B, = input_words.shape
    V, D = in_embed.shape

    info = plsc.get_sparse_core_info()
    nc, ns, L = info.num_cores, info.num_subcores, info.num_lanes
    nw = nc * ns
    bpw = B // nw

    ie3 = in_embed.reshape(V // 8, 8, D)
    oe3 = out_embed.reshape(V // 8, 8, D)

    mesh = plsc.VectorSubcoreMesh(core_axis_name="c", subcore_axis_name="s")

    @functools.partial(
        pl.kernel,
        mesh=mesh,
        out_type=jax.ShapeDtypeStruct((2, B // 8, 8, D), jnp.float32),
        scratch_types=[
            pltpu.VMEM((bpw,), jnp.int32),
            pltpu.VMEM((bpw,), jnp.int32),
            pltpu.VMEM((bpw // 8, 8, D), jnp.float32),
            pltpu.SemaphoreType.DMA,
        ],
    )
    def _gather2(iw_hbm, ow_hbm, ie_hbm, oe_hbm, out_hbm,
                 idx0, idx1, buf, sem):
        wid = lax.axis_index("s") * nc + lax.axis_index("c")
        base = wid * bpw
        pltpu.sync_copy(iw_hbm.at[pl.ds(base, bpw)], idx0)
        pltpu.sync_copy(ow_hbm.at[pl.ds(base, bpw)], idx1)

        for table, idx, out_row in ((ie_hbm, idx0, 0), (oe_hbm, idx1, 1)):
            def issue(k, _, table=table, idx=idx):
                v = idx[pl.ds(k * L, L)]
                tv = v >> 3
                sv = v & 7
                for j in range(L):
                    r = k * L + j
                    pltpu.async_copy(
                        table.at[tv[j], pl.ds(sv[j], 1)],
                        buf.at[r // 8, pl.ds(r % 8, 1)],
                        sem)
                return 0
            lax.fori_loop(0, bpw // L, issue, 0)
            pltpu.make_async_copy(
                ie_hbm.at[pl.ds(0, bpw // 8)], buf, sem).wait()
            pltpu.sync_copy(
                buf, out_hbm.at[out_row, pl.ds(base // 8, bpw // 8)])

    out = _gather2(input_words, output_words, ie3, oe3)
    return out.reshape(2, B, D)
